# baseline (speedup 1.0000x reference)
import functools
import math

import jax
import jax.numpy as jnp
from jax import lax
from jax.experimental import pallas as pl
from jax.experimental.pallas import tpu as pltpu
from jax.experimental.pallas import tpu_sc as plsc

D_MODEL = 1024
LANES = 16
NUM_WORKERS = 32
CHUNK = 8
SCALE = math.sqrt(D_MODEL)


def _emb_body(batch, seq_len, ids_hbm, table_hbm, pe_hbm, out_hbm,
              idx_v, tok0, tok1, tok2, tok3, pe0, pe1,
              res0, res1, res2, res3,
              st0, st1, st2, st3, sp0, sp1, sw0, sw1, sw2, sw3):
    toks = (tok0, tok1, tok2, tok3)
    pes = (pe0, pe1)
    ress = (res0, res1, res2, res3)
    sts = (st0, st1, st2, st3)
    sps = (sp0, sp1)
    sws = (sw0, sw1, sw2, sw3)

    pos_per_worker = seq_len // NUM_WORKERS
    n_pc = pos_per_worker // CHUNK
    steps = n_pc * batch

    wid = lax.axis_index("s") * 2 + lax.axis_index("c")
    wpos = wid * pos_per_worker

    def gather_copy(pc, bb, tb):
        return pltpu.make_async_copy(
            table_hbm.at[idx_v.at[pl.ds(bb * pos_per_worker + pc * CHUNK,
                                        CHUNK)]],
            toks[tb], sts[tb])

    def pe_copy(pc, pb):
        return pltpu.make_async_copy(
            pe_hbm.at[pl.ds(wpos + pc * CHUNK, CHUNK)], pes[pb], sps[pb])

    def write_copy(pc, bb, ob):
        return pltpu.make_async_copy(
            ress[ob], out_hbm.at[pl.ds(bb * seq_len + wpos + pc * CHUNK,
                                       CHUNK)], sws[ob])

    def idx_copy(b, sem):
        return pltpu.make_async_copy(
            ids_hbm.at[pl.ds(b * seq_len + wpos, pos_per_worker)],
            idx_v.at[pl.ds(b * pos_per_worker, pos_per_worker)], sem)

    pe_copy(0, 0).start()
    idx_sems = (st0, st1, st2, sw0)
    for b in range(batch):
        idx_copy(b, idx_sems[b]).start()
    for b in range(batch):
        idx_copy(b, idx_sems[b]).wait()
    for g in range(3):
        gather_copy(g // batch, g % batch, g).start()

    def outer(i, _):
        for j in range(2 * batch):
            pc = 2 * i + j // batch
            bb = j % batch
            g = 2 * batch * i + j
            tb = j % 4
            ob = j % 4
            pb = (j // batch) % 2

            if bb == 0:
                pe_copy(pc, pb).wait()

            @pl.when(g >= 4)
            def _():
                write_copy(pc - 1, bb, ob).wait()
            gather_copy(pc, bb, tb).wait()

            npc = 2 * i + (j + 3) // batch
            nbb = (j + 3) % batch
            ntb = (j + 3) % 4

            @pl.when(g + 3 < steps)
            def _():
                gather_copy(npc, nbb, ntb).start()

            def rows(r, _):
                for c in range(D_MODEL // LANES):
                    sl = pl.ds(c * LANES, LANES)
                    ress[ob][r, sl] = (toks[tb][r, sl] * SCALE
                                       + pes[pb][r, sl])
                return 0

            lax.fori_loop(0, CHUNK, rows, 0)
            write_copy(pc, bb, ob).start()

            if bb == 2:
                @pl.when(pc + 1 < n_pc)
                def _():
                    pe_copy(pc + 1, 1 - pb).start()
        return 0

    lax.fori_loop(0, steps // (2 * batch), outer, 0)

    for j in range(4):
        write_copy(n_pc - 1, j, j).wait()


@jax.jit
def kernel(token_ids, W_tok, pe):
    batch, seq_len = token_ids.shape
    n_rows = batch * seq_len
    ids = token_ids.reshape(-1).astype(jnp.int32)
    rows_per_worker = n_rows // NUM_WORKERS

    mesh = plsc.VectorSubcoreMesh(core_axis_name="c", subcore_axis_name="s")
    body = functools.partial(_emb_body, batch, seq_len)
    out = pl.kernel(
        body,
        mesh=mesh,
        out_type=jax.ShapeDtypeStruct((n_rows, D_MODEL), jnp.float32),
        scratch_types=(
            [pltpu.VMEM((rows_per_worker,), jnp.int32)]
            + [pltpu.VMEM((CHUNK, D_MODEL), jnp.float32) for _ in range(10)]
            + [pltpu.SemaphoreType.DMA for _ in range(10)]
        ),
    )(ids, W_tok, pe)
    return out.reshape(batch, seq_len, D_MODEL)
